# baseline (speedup 1.0000x reference)
import functools

import jax
import jax.numpy as jnp
from jax import lax
from jax.experimental import pallas as pl
from jax.experimental.pallas import tpu as pltpu
from jax.experimental.pallas import tpu_sc as plsc

N_NODES = 10000
N_EDGES = 320000
D_FEAT = 128

NC = 2
NS = 16

DHALF = D_FEAT // NC
E_PER_TILE = N_EDGES // NS
BATCH = 80
NBATCH = E_PER_TILE // BATCH
RING = 2
NGROUP = NBATCH // RING
ROWS_PER_TILE = N_NODES // NS


def _sc_kernel(xl, xr, src3, dst3, zeros, out,
               xs, acc, idx_s, idx_d, rows, gsem, ssem):
  c = lax.axis_index("c")
  s = lax.axis_index("s")
  r0 = s * ROWS_PER_TILE

  @pl.when(c == 0)
  def _():
    pltpu.async_copy(xl.at[pl.ds(r0, ROWS_PER_TILE)],
                     xs.at[pl.ds(r0, ROWS_PER_TILE)], ssem[0]).wait()

  @pl.when(c == 1)
  def _():
    pltpu.async_copy(xr.at[pl.ds(r0, ROWS_PER_TILE)],
                     xs.at[pl.ds(r0, ROWS_PER_TILE)], ssem[0]).wait()

  z = pltpu.async_copy(zeros, acc.at[pl.ds(r0, ROWS_PER_TILE)], gsem[0])
  a = pltpu.async_copy(src3.at[s], idx_s, gsem[1])
  b = pltpu.async_copy(dst3.at[s], idx_d, ssem[1])
  z.wait()
  a.wait()
  b.wait()
  plsc.subcore_barrier()

  def group(base, n):
    gd = [
        pltpu.async_copy(xs.at[idx_s.at[base + u]], rows[u], gsem[u])
        for u in range(n)
    ]
    sd = []
    for u in range(n):
      gd[u].wait()
      sd.append(
          pltpu.async_copy(rows[u], acc.at[idx_d.at[base + u]], ssem[u],
                           add=True))
    for u in range(n):
      sd[u].wait()

  def step(g, carry):
    group(g * RING, RING)
    return carry
  lax.fori_loop(0, NGROUP, step, 0)

  plsc.subcore_barrier()

  pltpu.sync_copy(
      acc.at[pl.ds(r0, ROWS_PER_TILE)],
      out.at[pl.ds(r0, ROWS_PER_TILE), pl.ds(c * DHALF, DHALF)],
  )


@jax.jit
def _path_add(xl, xr, src3, dst3, zeros):
  mesh = plsc.VectorSubcoreMesh(core_axis_name="c", subcore_axis_name="s")
  return pl.kernel(
      _sc_kernel,
      out_type=jax.ShapeDtypeStruct((N_NODES, D_FEAT), jnp.float32),
      mesh=mesh,
      scratch_types=[
          pltpu.VMEM_SHARED((N_NODES, DHALF), jnp.float32),
          pltpu.VMEM_SHARED((N_NODES, DHALF), jnp.float32),
          pltpu.VMEM((NBATCH, BATCH), jnp.int32),
          pltpu.VMEM((NBATCH, BATCH), jnp.int32),
          [pltpu.VMEM((BATCH, DHALF), jnp.float32)
           for _ in range(RING)],
          [pltpu.SemaphoreType.DMA for _ in range(RING)],
          [pltpu.SemaphoreType.DMA for _ in range(RING)],
      ],
      compiler_params=pltpu.CompilerParams(use_tc_tiling_on_sc=False),
      name="path_add_sc",
  )(xl, xr, src3, dst3, zeros)


def kernel(x, edge_index):
  xl = x[:, :DHALF]
  xr = x[:, DHALF:]
  src3 = edge_index[0].reshape(NS, NBATCH, BATCH)
  dst3 = edge_index[1].reshape(NS, NBATCH, BATCH)
  zeros = jnp.zeros((ROWS_PER_TILE, DHALF), jnp.float32)
  return _path_add(xl, xr, src3, dst3, zeros)


# speedup vs baseline: 1.1889x; 1.1889x over previous
import functools

import jax
import jax.numpy as jnp
from jax import lax
from jax.experimental import pallas as pl
from jax.experimental.pallas import tpu as pltpu
from jax.experimental.pallas import tpu_sc as plsc

N_NODES = 10000
N_EDGES = 320000
D_FEAT = 128

NC = 2
NS = 16

DHALF = D_FEAT // NC
E_PER_TILE = N_EDGES // NS
BATCH = 128
NBATCH = 157
E_PAD_TILE = NBATCH * BATCH
RING = 6
NGROUP = NBATCH // RING
NLEFT = NBATCH - NGROUP * RING
N_PAD = 10240
ROWS_PER_TILE = N_PAD // NS
LAST_ROWS = N_NODES - 15 * ROWS_PER_TILE


def _sc_kernel(xl, xr, src3, dst3, zeros, out,
               acc, idx_s, idx_d, rows, gsem, ssem):
  c = lax.axis_index("c")
  s = lax.axis_index("s")
  r0 = s * ROWS_PER_TILE

  z = pltpu.async_copy(zeros, acc.at[pl.ds(r0, ROWS_PER_TILE)], gsem[0])
  a = pltpu.async_copy(src3.at[s], idx_s, gsem[1])
  b = pltpu.async_copy(dst3.at[s], idx_d, gsem[2])
  z.wait()
  a.wait()
  b.wait()
  plsc.subcore_barrier()

  def body(xh):
    def group(base, n):
      gd = [
          pltpu.async_copy(xh.at[idx_s.at[base + u]], rows[u], gsem[u])
          for u in range(n)
      ]
      sd = []
      for u in range(n):
        gd[u].wait()
        sd.append(
            pltpu.async_copy(rows[u], acc.at[idx_d.at[base + u]], ssem[u],
                             add=True))
      for u in range(n):
        sd[u].wait()

    def step(g, carry):
      group(g * RING, RING)
      return carry
    lax.fori_loop(0, NGROUP, step, 0)
    if NLEFT:
      group(NGROUP * RING, NLEFT)

  @pl.when(c == 0)
  def _():
    body(xl)

  @pl.when(c == 1)
  def _():
    body(xr)

  plsc.subcore_barrier()

  @pl.when(s < NS - 1)
  def _():
    pltpu.sync_copy(
        acc.at[pl.ds(r0, ROWS_PER_TILE)],
        out.at[pl.ds(r0, ROWS_PER_TILE), pl.ds(c * DHALF, DHALF)],
    )

  @pl.when(s == NS - 1)
  def _():
    pltpu.sync_copy(
        acc.at[pl.ds(r0, LAST_ROWS)],
        out.at[pl.ds(r0, LAST_ROWS), pl.ds(c * DHALF, DHALF)],
    )


@jax.jit
def _path_add(xl, xr, src3, dst3, zeros):
  mesh = plsc.VectorSubcoreMesh(core_axis_name="c", subcore_axis_name="s")
  return pl.kernel(
      _sc_kernel,
      out_type=jax.ShapeDtypeStruct((N_NODES, D_FEAT), jnp.float32),
      mesh=mesh,
      scratch_types=[
          pltpu.VMEM_SHARED((N_PAD, DHALF), jnp.float32),
          pltpu.VMEM((NBATCH, BATCH), jnp.int32),
          pltpu.VMEM((NBATCH, BATCH), jnp.int32),
          [pltpu.VMEM((BATCH, DHALF), jnp.float32)
           for _ in range(RING)],
          [pltpu.SemaphoreType.DMA for _ in range(RING)],
          [pltpu.SemaphoreType.DMA for _ in range(RING)],
      ],
      compiler_params=pltpu.CompilerParams(use_tc_tiling_on_sc=False),
      name="path_add_sc",
  )(xl, xr, src3, dst3, zeros)


def kernel(x, edge_index):
  xl = x[:, :DHALF]
  xr = x[:, DHALF:]
  npad = NS * E_PAD_TILE - N_EDGES
  src3 = jnp.concatenate(
      [edge_index[0], jnp.zeros((npad,), edge_index.dtype)]
  ).reshape(NS, NBATCH, BATCH)
  dst3 = jnp.concatenate(
      [edge_index[1], jnp.full((npad,), N_NODES, edge_index.dtype)]
  ).reshape(NS, NBATCH, BATCH)
  zeros = jnp.zeros((ROWS_PER_TILE, DHALF), jnp.float32)
  return _path_add(xl, xr, src3, dst3, zeros)


# speedup vs baseline: 1.3449x; 1.1312x over previous
import functools

import jax
import jax.numpy as jnp
from jax import lax
from jax.experimental import pallas as pl
from jax.experimental.pallas import tpu as pltpu
from jax.experimental.pallas import tpu_sc as plsc

N_NODES = 10000
N_EDGES = 320000
D_FEAT = 128

NC = 2
NS = 16

DHALF = D_FEAT // NC
E_PER_TILE = N_EDGES // NS
BATCH = 200
NBATCH = E_PER_TILE // BATCH
RING = 3
NGROUP = NBATCH // RING
NLEFT = NBATCH - NGROUP * RING
N_PAD = 10240
ROWS_PER_TILE = N_PAD // NS
LAST_ROWS = N_NODES - 15 * ROWS_PER_TILE


def _sc_kernel(xl, xr, src3, dst3, zeros, out,
               acc, idx_s, idx_d, rows, gsem, ssem):
  c = lax.axis_index("c")
  s = lax.axis_index("s")
  r0 = s * ROWS_PER_TILE

  z = pltpu.async_copy(zeros, acc.at[pl.ds(r0, ROWS_PER_TILE)], gsem[0])
  a = pltpu.async_copy(src3.at[s], idx_s, gsem[1])
  b = pltpu.async_copy(dst3.at[s], idx_d, gsem[2])
  z.wait()
  a.wait()
  b.wait()
  plsc.subcore_barrier()

  def body(xh):
    def group(base, n):
      gd = [
          pltpu.async_copy(xh.at[idx_s.at[base + u]], rows[u], gsem[u])
          for u in range(n)
      ]
      sd = []
      for u in range(n):
        gd[u].wait()
        sd.append(
            pltpu.async_copy(rows[u], acc.at[idx_d.at[base + u]], ssem[u],
                             add=True))
      for u in range(n):
        sd[u].wait()

    def step(g, carry):
      group(g * RING, RING)
      return carry
    lax.fori_loop(0, NGROUP, step, 0)
    if NLEFT:
      group(NGROUP * RING, NLEFT)

  @pl.when(c == 0)
  def _():
    body(xl)

  @pl.when(c == 1)
  def _():
    body(xr)

  plsc.subcore_barrier()

  @pl.when(s < NS - 1)
  def _():
    pltpu.sync_copy(
        acc.at[pl.ds(r0, ROWS_PER_TILE)],
        out.at[pl.ds(r0, ROWS_PER_TILE), pl.ds(c * DHALF, DHALF)],
    )

  @pl.when(s == NS - 1)
  def _():
    pltpu.sync_copy(
        acc.at[pl.ds(r0, LAST_ROWS)],
        out.at[pl.ds(r0, LAST_ROWS), pl.ds(c * DHALF, DHALF)],
    )


@jax.jit
def _path_add(xl, xr, src3, dst3, zeros):
  mesh = plsc.VectorSubcoreMesh(core_axis_name="c", subcore_axis_name="s")
  return pl.kernel(
      _sc_kernel,
      out_type=jax.ShapeDtypeStruct((N_NODES, D_FEAT), jnp.float32),
      mesh=mesh,
      scratch_types=[
          pltpu.VMEM_SHARED((N_PAD, DHALF), jnp.float32),
          pltpu.VMEM((NBATCH, BATCH), jnp.int32),
          pltpu.VMEM((NBATCH, BATCH), jnp.int32),
          [pltpu.VMEM((BATCH, DHALF), jnp.float32)
           for _ in range(RING)],
          [pltpu.SemaphoreType.DMA for _ in range(RING)],
          [pltpu.SemaphoreType.DMA for _ in range(RING)],
      ],
      compiler_params=pltpu.CompilerParams(use_tc_tiling_on_sc=False),
      name="path_add_sc",
  )(xl, xr, src3, dst3, zeros)


def kernel(x, edge_index):
  xl = x[:, :DHALF]
  xr = x[:, DHALF:]
  src3 = edge_index[0].reshape(NS, NBATCH, BATCH)
  dst3 = edge_index[1].reshape(NS, NBATCH, BATCH)
  zeros = jnp.zeros((ROWS_PER_TILE, DHALF), jnp.float32)
  return _path_add(xl, xr, src3, dst3, zeros)


# speedup vs baseline: 1.6613x; 1.2352x over previous
import functools

import jax
import jax.numpy as jnp
from jax import lax
from jax.experimental import pallas as pl
from jax.experimental.pallas import tpu as pltpu
from jax.experimental.pallas import tpu_sc as plsc

N_NODES = 10000
N_EDGES = 320000
D_FEAT = 128

NC = 2
NS = 16

DHALF = D_FEAT // NC
E_PER_TILE = N_EDGES // NS
BATCH = 160
NBATCH = E_PER_TILE // BATCH
HALF = 2
RING = 2 * HALF
NITER = NBATCH // RING
NLEFT = NBATCH - NITER * RING
N_PAD = 10240
ROWS_PER_TILE = N_PAD // NS
LAST_ROWS = N_NODES - 15 * ROWS_PER_TILE


def _sc_kernel(xl, xr, src3, dst3, zeros, out,
               acc, idx_s, idx_d, rows, gsem, ssem):
  c = lax.axis_index("c")
  s = lax.axis_index("s")
  r0 = s * ROWS_PER_TILE

  z = pltpu.async_copy(zeros, acc.at[pl.ds(r0, ROWS_PER_TILE)], gsem[0])
  a = pltpu.async_copy(src3.at[s], idx_s, gsem[1])
  b = pltpu.async_copy(dst3.at[s], idx_d, gsem[2])
  z.wait()
  a.wait()
  b.wait()
  plsc.subcore_barrier()

  def body(xh):
    def issue_gather(b, u):
      return pltpu.async_copy(xh.at[idx_s.at[b]], rows[u], gsem[u])

    def wait_gather(b, u):
      pltpu.make_async_copy(xh.at[idx_s.at[b]], rows[u], gsem[u]).wait()

    def issue_scatter(b, u):
      return pltpu.async_copy(rows[u], acc.at[idx_d.at[b]], ssem[u],
                              add=True)

    def wait_scatter(b, u):
      pltpu.make_async_copy(rows[u], acc.at[idx_d.at[b]], ssem[u]).wait()

    for u in range(HALF):
      issue_gather(u, u)

    def step(q, carry):
      bA = q * RING
      bB = bA + HALF
      for u in range(HALF):
        issue_gather(bB + u, HALF + u)
      for u in range(HALF):
        wait_gather(bA + u, u)
        issue_scatter(bA + u, u)
      for u in range(HALF):
        wait_scatter(bA + u, u)

      @pl.when(q < NITER - 1)
      def _():
        for u in range(HALF):
          issue_gather(bA + RING + u, u)

      for u in range(HALF):
        wait_gather(bB + u, HALF + u)
        issue_scatter(bB + u, HALF + u)
      for u in range(HALF):
        wait_scatter(bB + u, HALF + u)
      return carry
    lax.fori_loop(0, NITER, step, 0)

    for u in range(NLEFT):
      b = NITER * RING + u
      issue_gather(b, u).wait()
      issue_scatter(b, u).wait()

  @pl.when(c == 0)
  def _():
    body(xl)

  @pl.when(c == 1)
  def _():
    body(xr)

  plsc.subcore_barrier()

  @pl.when(s < NS - 1)
  def _():
    pltpu.sync_copy(
        acc.at[pl.ds(r0, ROWS_PER_TILE)],
        out.at[pl.ds(r0, ROWS_PER_TILE), pl.ds(c * DHALF, DHALF)],
    )

  @pl.when(s == NS - 1)
  def _():
    pltpu.sync_copy(
        acc.at[pl.ds(r0, LAST_ROWS)],
        out.at[pl.ds(r0, LAST_ROWS), pl.ds(c * DHALF, DHALF)],
    )


@jax.jit
def _path_add(xl, xr, src3, dst3, zeros):
  mesh = plsc.VectorSubcoreMesh(core_axis_name="c", subcore_axis_name="s")
  return pl.kernel(
      _sc_kernel,
      out_type=jax.ShapeDtypeStruct((N_NODES, D_FEAT), jnp.float32),
      mesh=mesh,
      scratch_types=[
          pltpu.VMEM_SHARED((N_PAD, DHALF), jnp.float32),
          pltpu.VMEM((NBATCH, BATCH), jnp.int32),
          pltpu.VMEM((NBATCH, BATCH), jnp.int32),
          [pltpu.VMEM((BATCH, DHALF), jnp.float32)
           for _ in range(RING)],
          [pltpu.SemaphoreType.DMA for _ in range(RING)],
          [pltpu.SemaphoreType.DMA for _ in range(RING)],
      ],
      compiler_params=pltpu.CompilerParams(use_tc_tiling_on_sc=False),
      name="path_add_sc",
  )(xl, xr, src3, dst3, zeros)


def kernel(x, edge_index):
  xl = x[:, :DHALF]
  xr = x[:, DHALF:]
  src3 = edge_index[0].reshape(NS, NBATCH, BATCH)
  dst3 = edge_index[1].reshape(NS, NBATCH, BATCH)
  zeros = jnp.zeros((ROWS_PER_TILE, DHALF), jnp.float32)
  return _path_add(xl, xr, src3, dst3, zeros)


# speedup vs baseline: 1.7328x; 1.0431x over previous
import functools

import jax
import jax.numpy as jnp
from jax import lax
from jax.experimental import pallas as pl
from jax.experimental.pallas import tpu as pltpu
from jax.experimental.pallas import tpu_sc as plsc

N_NODES = 10000
N_EDGES = 320000
D_FEAT = 128

NC = 2
NS = 16

DHALF = D_FEAT // NC
E_PER_TILE = N_EDGES // NS
BATCH = 160
NBATCH = E_PER_TILE // BATCH
HALF = 2
RING = 2 * HALF
NITER = NBATCH // RING
NLEFT = NBATCH - NITER * RING
N_PAD = 10240
ROWS_PER_TILE = N_PAD // NS
LAST_ROWS = N_NODES - 15 * ROWS_PER_TILE


def _sc_kernel(x2, src4, dst3, out,
               acc, idx_s, idx_d, rows, gsem, ssem):
  c = lax.axis_index("c")
  s = lax.axis_index("s")
  r0 = s * ROWS_PER_TILE

  a = pltpu.async_copy(src4.at[c, s], idx_s, gsem[0])
  b = pltpu.async_copy(dst3.at[s], idx_d, gsem[1])

  zv = jnp.zeros((16,), jnp.float32)
  def zstep(i, carry):
    for j in range(DHALF // 16):
      rows[0][i, pl.ds(j * 16, 16)] = zv
    return carry
  lax.fori_loop(0, BATCH, zstep, 0)
  for k in range(ROWS_PER_TILE // BATCH):
    pltpu.sync_copy(rows[0], acc.at[pl.ds(r0 + k * BATCH, BATCH)])
  a.wait()
  b.wait()
  plsc.subcore_barrier()

  def body(xh):
    def issue_gather(b, u):
      return pltpu.async_copy(xh.at[idx_s.at[b]], rows[u], gsem[u])

    def wait_gather(b, u):
      pltpu.make_async_copy(xh.at[idx_s.at[b]], rows[u], gsem[u]).wait()

    def issue_scatter(b, u):
      return pltpu.async_copy(rows[u], acc.at[idx_d.at[b]], ssem[u],
                              add=True)

    def wait_scatter(b, u):
      pltpu.make_async_copy(rows[u], acc.at[idx_d.at[b]], ssem[u]).wait()

    for u in range(HALF):
      issue_gather(u, u)

    def step(q, carry):
      bA = q * RING
      bB = bA + HALF
      for u in range(HALF):
        issue_gather(bB + u, HALF + u)
      for u in range(HALF):
        wait_gather(bA + u, u)
        issue_scatter(bA + u, u)
      for u in range(HALF):
        wait_scatter(bA + u, u)

      @pl.when(q < NITER - 1)
      def _():
        for u in range(HALF):
          issue_gather(bA + RING + u, u)

      for u in range(HALF):
        wait_gather(bB + u, HALF + u)
        issue_scatter(bB + u, HALF + u)
      for u in range(HALF):
        wait_scatter(bB + u, HALF + u)
      return carry
    lax.fori_loop(0, NITER, step, 0)

    for u in range(NLEFT):
      b = NITER * RING + u
      issue_gather(b, u).wait()
      issue_scatter(b, u).wait()

  body(x2)

  plsc.subcore_barrier()

  @pl.when(s < NS - 1)
  def _():
    pltpu.sync_copy(
        acc.at[pl.ds(r0, ROWS_PER_TILE)],
        out.at[pl.ds(r0, ROWS_PER_TILE), pl.ds(c * DHALF, DHALF)],
    )

  @pl.when(s == NS - 1)
  def _():
    pltpu.sync_copy(
        acc.at[pl.ds(r0, LAST_ROWS)],
        out.at[pl.ds(r0, LAST_ROWS), pl.ds(c * DHALF, DHALF)],
    )


@jax.jit
def _path_add(x2, src4, dst3):
  mesh = plsc.VectorSubcoreMesh(core_axis_name="c", subcore_axis_name="s")
  return pl.kernel(
      _sc_kernel,
      out_type=jax.ShapeDtypeStruct((N_NODES, D_FEAT), jnp.float32),
      mesh=mesh,
      scratch_types=[
          pltpu.VMEM_SHARED((N_PAD, DHALF), jnp.float32),
          pltpu.VMEM((NBATCH, BATCH), jnp.int32),
          pltpu.VMEM((NBATCH, BATCH), jnp.int32),
          [pltpu.VMEM((BATCH, DHALF), jnp.float32)
           for _ in range(RING)],
          [pltpu.SemaphoreType.DMA for _ in range(RING)],
          [pltpu.SemaphoreType.DMA for _ in range(RING)],
      ],
      compiler_params=pltpu.CompilerParams(use_tc_tiling_on_sc=False),
      name="path_add_sc",
  )(x2, src4, dst3)


def kernel(x, edge_index):
  x2 = x.reshape(NC * N_NODES, DHALF)
  src2 = edge_index[0] * 2
  src4 = jnp.stack([src2, src2 + 1]).reshape(NC, NS, NBATCH, BATCH)
  dst3 = edge_index[1].reshape(NS, NBATCH, BATCH)
  return _path_add(x2, src4, dst3)
